# Initial kernel scaffold; baseline (speedup 1.0000x reference)
import functools

import jax
import jax.numpy as jnp
from jax import lax
from jax.experimental import pallas as pl
from jax.experimental.pallas import tpu as pltpu
from jax.experimental.pallas import tpu_sc as plsc

NE = 8
CAP = 1280
NTOK = 8192
D = 2048
BLK = 256
NBLK = NTOK // BLK
NROWS = NE * CAP
NW = 32
RPT = NROWS // NW
CHUNK = 16
NCH = RPT // CHUNK
DROP = NROWS


def _router_body(x_ref, w_ref, b_ref, slot_ref, counts_ref):
    i = pl.program_id(0)

    @pl.when(i == 0)
    def _():
        counts_ref[...] = jnp.zeros((1, NE), jnp.float32)

    logits = (
        jnp.dot(x_ref[...], w_ref[...], preferred_element_type=jnp.float32)
        + b_ref[...]
    )
    iota_e = lax.broadcasted_iota(jnp.int32, (BLK, NE), 1)
    mx = jnp.max(logits, axis=1, keepdims=True)
    routes = jnp.min(
        jnp.where(logits == mx, iota_e, NE), axis=1, keepdims=True
    )
    one_hot = (iota_e == routes).astype(jnp.float32)
    row_i = lax.broadcasted_iota(jnp.int32, (BLK, BLK), 0)
    col_i = lax.broadcasted_iota(jnp.int32, (BLK, BLK), 1)
    tril = (row_i >= col_i).astype(jnp.float32)
    pos_incl = jnp.dot(tril, one_hot, preferred_element_type=jnp.float32)
    counts = counts_ref[...]
    pos_tok = jnp.sum(one_hot * (pos_incl + counts), axis=1, keepdims=True) - 1.0
    pos_i = pos_tok.astype(jnp.int32)
    keep = pos_i < CAP
    slot = jnp.where(keep, routes * CAP + pos_i, DROP)
    slot_ref[...] = slot.reshape(1, BLK, 1)
    counts_ref[...] = counts + jnp.sum(one_hot, axis=0, keepdims=True)


_router = pl.pallas_call(
    _router_body,
    grid=(NBLK,),
    in_specs=[
        pl.BlockSpec((BLK, D), lambda i: (i, 0)),
        pl.BlockSpec((D, NE), lambda i: (0, 0)),
        pl.BlockSpec((1, NE), lambda i: (0, 0)),
    ],
    out_specs=pl.BlockSpec((1, BLK, 1), lambda i: (i, 0, 0)),
    out_shape=jax.ShapeDtypeStruct((NBLK, BLK, 1), jnp.int32),
    scratch_shapes=[pltpu.VMEM((1, NE), jnp.float32)],
)


def _dispatch_body(x_hbm, slots_hbm, zeros_hbm, out_hbm,
                   slots_v, perm_v, idx_v, stage_v, zero_v, sem):
    wid = lax.axis_index("s") * 2 + lax.axis_index("c")
    lo = wid * RPT
    pltpu.sync_copy(slots_hbm, slots_v)
    pltpu.sync_copy(zeros_hbm, zero_v)

    def initp(j, carry):
        perm_v[pl.ds(j * 16, 16)] = jnp.zeros((16,), jnp.int32)
        return carry

    lax.fori_loop(0, RPT // 16, initp, 0)

    lov = jnp.full((16,), lo, jnp.int32)

    def build(j, fcar):
        s = slots_v[pl.ds(j * 16, 16)]
        tok = lax.iota(jnp.int32, 16) + j * 16
        rel = s - lov
        m = (rel >= 0) & (rel < RPT)
        plsc.store_scatter(perm_v, [rel], tok, mask=m)
        return fcar + plsc.all_reduce_population_count(m)

    fvec = lax.fori_loop(0, NTOK // 16, build, jnp.zeros((16,), jnp.int32))
    f = jnp.max(fvec, axis=0)

    def chunk(c, carry):
        base = c * CHUNK

        @pl.when(base < f)
        def _():
            idx_v[...] = perm_v[pl.ds(base, CHUNK)]
            pltpu.async_copy(x_hbm.at[idx_v], stage_v, sem).wait()
            for r in range(CHUNK):
                @pl.when(base + r >= f)
                def _():
                    pltpu.sync_copy(zero_v.at[pl.ds(0, 1)],
                                    stage_v.at[pl.ds(r, 1)])
            pltpu.sync_copy(stage_v, out_hbm.at[pl.ds(lo + base, CHUNK)])

        @pl.when(base >= f)
        def _():
            pltpu.sync_copy(zero_v, out_hbm.at[pl.ds(lo + base, CHUNK)])

        return carry

    lax.fori_loop(0, NCH, chunk, 0)


_dispatch = functools.partial(
    pl.kernel,
    out_type=jax.ShapeDtypeStruct((NROWS, D), jnp.float32),
    mesh=plsc.VectorSubcoreMesh(core_axis_name="c", subcore_axis_name="s"),
    scratch_types=[
        pltpu.VMEM((NTOK,), jnp.int32),
        pltpu.VMEM((RPT,), jnp.int32),
        pltpu.VMEM((CHUNK,), jnp.int32),
        pltpu.VMEM((CHUNK, D), jnp.float32),
        pltpu.VMEM((CHUNK, D), jnp.float32),
        pltpu.SemaphoreType.DMA,
    ],
)(_dispatch_body)


def kernel(x, W, b):
    xf = x.reshape(-1, D)
    slots = _router(xf, W, b.reshape(1, NE)).reshape(NTOK)
    zrows = jnp.zeros((CHUNK, D), jnp.float32)
    return _dispatch(xf, slots, zrows)


# speedup vs baseline: 2.0356x; 2.0356x over previous
import functools

import jax
import jax.numpy as jnp
from jax import lax
from jax.experimental import pallas as pl
from jax.experimental.pallas import tpu as pltpu
from jax.experimental.pallas import tpu_sc as plsc

NE = 8
CAP = 1280
NTOK = 8192
D = 2048
BLK = 256
NBLK = NTOK // BLK
NROWS = NE * CAP
NW = 32
RPT = NROWS // NW
CHUNK = 16
NCH = RPT // CHUNK
DROP = NROWS


def _router_body(x_ref, w_ref, b_ref, slot_ref, counts_ref):
    i = pl.program_id(0)

    @pl.when(i == 0)
    def _():
        counts_ref[...] = jnp.zeros((1, NE), jnp.float32)

    logits = (
        jnp.dot(x_ref[...], w_ref[...], preferred_element_type=jnp.float32)
        + b_ref[...]
    )
    iota_e = lax.broadcasted_iota(jnp.int32, (BLK, NE), 1)
    mx = jnp.max(logits, axis=1, keepdims=True)
    routes = jnp.min(
        jnp.where(logits == mx, iota_e, NE), axis=1, keepdims=True
    )
    one_hot = (iota_e == routes).astype(jnp.float32)
    row_i = lax.broadcasted_iota(jnp.int32, (BLK, BLK), 0)
    col_i = lax.broadcasted_iota(jnp.int32, (BLK, BLK), 1)
    tril = (row_i >= col_i).astype(jnp.float32)
    pos_incl = jnp.dot(tril, one_hot, preferred_element_type=jnp.float32)
    counts = counts_ref[...]
    pos_tok = jnp.sum(one_hot * (pos_incl + counts), axis=1, keepdims=True) - 1.0
    pos_i = pos_tok.astype(jnp.int32)
    keep = pos_i < CAP
    slot = jnp.where(keep, routes * CAP + pos_i, DROP)
    slot_ref[...] = slot.reshape(1, BLK, 1)
    counts_ref[...] = counts + jnp.sum(one_hot, axis=0, keepdims=True)


_router = pl.pallas_call(
    _router_body,
    grid=(NBLK,),
    in_specs=[
        pl.BlockSpec((BLK, D), lambda i: (i, 0)),
        pl.BlockSpec((D, NE), lambda i: (0, 0)),
        pl.BlockSpec((1, NE), lambda i: (0, 0)),
    ],
    out_specs=pl.BlockSpec((1, BLK, 1), lambda i: (i, 0, 0)),
    out_shape=jax.ShapeDtypeStruct((NBLK, BLK, 1), jnp.int32),
    scratch_shapes=[pltpu.VMEM((1, NE), jnp.float32)],
)


def _dispatch_body(x_hbm, slots_hbm, zeros_hbm, out_hbm,
                   slots_v, perm_v, stage_a, stage_b, zero_v, sem_g, sem_w):
    wid = lax.axis_index("s") * 2 + lax.axis_index("c")
    lo = wid * RPT
    pltpu.sync_copy(slots_hbm, slots_v)
    pltpu.sync_copy(zeros_hbm, zero_v)

    def initp(j, carry):
        perm_v[pl.ds(j * 16, 16)] = jnp.zeros((16,), jnp.int32)
        return carry

    lax.fori_loop(0, RPT // 16, initp, 0)

    lov = jnp.full((16,), lo, jnp.int32)

    def build(j, fcar):
        s = slots_v[pl.ds(j * 16, 16)]
        tok = lax.iota(jnp.int32, 16) + j * 16
        rel = s - lov
        m = (rel >= 0) & (rel < RPT)
        plsc.store_scatter(perm_v, [rel], tok, mask=m)
        return fcar + plsc.all_reduce_population_count(m)

    fvec = lax.fori_loop(0, NTOK // 16, build, jnp.zeros((16,), jnp.int32))
    f = jnp.max(fvec, axis=0)
    nfc = (f + CHUNK - 1) // CHUNK

    def issue_g(c, buf):
        idx_ref = perm_v.at[pl.ds(c * CHUNK, CHUNK)]
        pltpu.async_copy(x_hbm.at[idx_ref], buf, sem_g)

    def wait_g():
        pltpu.make_async_copy(x_hbm.at[perm_v.at[pl.ds(0, CHUNK)]], stage_a,
                              sem_g).wait()

    def issue_w(c, buf):
        @pl.when(c == nfc - 1)
        def _():
            def fixr(r, carry):
                pltpu.sync_copy(zeros_hbm.at[pl.ds(0, 1)],
                                buf.at[pl.ds(r, 1)])
                return carry

            lax.fori_loop(f - c * CHUNK, CHUNK, fixr, 0)

        pltpu.async_copy(buf, out_hbm.at[pl.ds(lo + c * CHUNK, CHUNK)], sem_w)

    def wait_w():
        pltpu.make_async_copy(zeros_hbm, stage_a, sem_w).wait()

    @pl.when(nfc > 0)
    def _():
        issue_g(0, stage_a)

    def body_a(p, carry):
        a = 2 * p
        b = 2 * p + 1

        @pl.when(a < nfc)
        def _():
            wait_g()

            @pl.when(a >= 1)
            def _():
                wait_w()

            @pl.when(b < nfc)
            def _():
                issue_g(b, stage_b)

            issue_w(a, stage_a)

        @pl.when(b < nfc)
        def _():
            wait_g()

            @pl.when(b + 1 < nfc)
            def _():
                issue_g(b + 1, stage_a)

            wait_w()
            issue_w(b, stage_b)

        return carry

    lax.fori_loop(0, (NCH + 1) // 2, body_a, 0)

    @pl.when(nfc > 0)
    def _():
        wait_w()

    def body_b(c, carry):
        pltpu.async_copy(zero_v, out_hbm.at[pl.ds(lo + c * CHUNK, CHUNK)],
                         sem_w)
        return carry

    lax.fori_loop(nfc, NCH, body_b, 0)

    def drain_b(c, carry):
        wait_w()
        return carry

    lax.fori_loop(nfc, NCH, drain_b, 0)


@functools.cache
def _make_dispatch():
    return functools.partial(
        pl.kernel,
        out_type=jax.ShapeDtypeStruct((NROWS, D), jnp.float32),
        mesh=plsc.VectorSubcoreMesh(core_axis_name="c", subcore_axis_name="s"),
        compiler_params=pltpu.CompilerParams(needs_layout_passes=False),
        scratch_types=[
            pltpu.VMEM((NTOK,), jnp.int32),
            pltpu.VMEM((512,), jnp.int32),
            pltpu.VMEM((CHUNK, D), jnp.float32),
            pltpu.VMEM((CHUNK, D), jnp.float32),
            pltpu.VMEM((CHUNK, D), jnp.float32),
            pltpu.SemaphoreType.DMA,
            pltpu.SemaphoreType.DMA,
        ],
    )(_dispatch_body)


def kernel(x, W, b):
    xf = x.reshape(-1, D)
    slots = _router(xf, W, b.reshape(1, NE)).reshape(NTOK)
    zrows = jnp.zeros((CHUNK, D), jnp.float32)
    return _make_dispatch()(xf, slots, zrows)


# speedup vs baseline: 2.1321x; 1.0474x over previous
import functools

import jax
import jax.numpy as jnp
from jax import lax
from jax.experimental import pallas as pl
from jax.experimental.pallas import tpu as pltpu
from jax.experimental.pallas import tpu_sc as plsc

NE = 8
CAP = 1280
NTOK = 8192
D = 2048
BLK = 1024
NBLK = NTOK // BLK
NROWS = NE * CAP
NW = 32
RPT = NROWS // NW
CHUNK = 16
NCH = RPT // CHUNK
DROP = NROWS


def _router_body(x_ref, w_ref, b_ref, slot_ref, counts_ref, tril_ref):
    i = pl.program_id(0)

    @pl.when(i == 0)
    def _():
        counts_ref[...] = jnp.zeros((1, NE), jnp.float32)
        row_i = lax.broadcasted_iota(jnp.int32, (BLK, BLK), 0)
        col_i = lax.broadcasted_iota(jnp.int32, (BLK, BLK), 1)
        tril_ref[...] = (row_i >= col_i).astype(jnp.float32)

    logits = (
        jnp.dot(x_ref[...], w_ref[...], preferred_element_type=jnp.float32)
        + b_ref[...]
    )
    iota_e = lax.broadcasted_iota(jnp.int32, (BLK, NE), 1)
    mx = jnp.max(logits, axis=1, keepdims=True)
    routes = jnp.min(
        jnp.where(logits == mx, iota_e, NE), axis=1, keepdims=True
    )
    one_hot = (iota_e == routes).astype(jnp.float32)
    pos_incl = jnp.dot(tril_ref[...], one_hot,
                       preferred_element_type=jnp.float32)
    counts = counts_ref[...]
    pos_tok = jnp.sum(one_hot * (pos_incl + counts), axis=1, keepdims=True) - 1.0
    pos_i = pos_tok.astype(jnp.int32)
    keep = pos_i < CAP
    slot = jnp.where(keep, routes * CAP + pos_i, DROP)
    slot_ref[...] = slot.reshape(1, BLK, 1)
    counts_ref[...] = counts + jnp.sum(one_hot, axis=0, keepdims=True)


_router = pl.pallas_call(
    _router_body,
    grid=(NBLK,),
    in_specs=[
        pl.BlockSpec((BLK, D), lambda i: (i, 0)),
        pl.BlockSpec((D, NE), lambda i: (0, 0)),
        pl.BlockSpec((1, NE), lambda i: (0, 0)),
    ],
    out_specs=pl.BlockSpec((1, BLK, 1), lambda i: (i, 0, 0)),
    out_shape=jax.ShapeDtypeStruct((NBLK, BLK, 1), jnp.int32),
    scratch_shapes=[
        pltpu.VMEM((1, NE), jnp.float32),
        pltpu.VMEM((BLK, BLK), jnp.float32),
    ],
)


def _dispatch_body(x_hbm, slots_hbm, zeros_hbm, out_hbm,
                   slots_v, perm_v, stage_a, stage_b, stage_c,
                   sem_ga, sem_gb, sem_gc, sem_wa, sem_wb, sem_wc):
    wid = lax.axis_index("s") * 2 + lax.axis_index("c")
    lo = wid * RPT
    bufs = (stage_a, stage_b, stage_c)
    gsems = (sem_ga, sem_gb, sem_gc)
    wsems = (sem_wa, sem_wb, sem_wc)
    pltpu.sync_copy(slots_hbm, slots_v)

    def initp(j, carry):
        perm_v[pl.ds(j * 16, 16)] = jnp.zeros((16,), jnp.int32)
        return carry

    lax.fori_loop(0, RPT // 16, initp, 0)

    lov = jnp.full((16,), lo, jnp.int32)

    def build(j, fcar):
        s = slots_v[pl.ds(j * 16, 16)]
        tok = lax.iota(jnp.int32, 16) + j * 16
        rel = s - lov
        m = (rel >= 0) & (rel < RPT)
        plsc.store_scatter(perm_v, [rel], tok, mask=m)
        return fcar + plsc.all_reduce_population_count(m)

    fvec = lax.fori_loop(0, NTOK // 16, build, jnp.zeros((16,), jnp.int32))
    f = jnp.max(fvec, axis=0)
    nfc = (f + CHUNK - 1) // CHUNK

    NBUF = 3

    def issue_g(c, k):
        idx_ref = perm_v.at[pl.ds(c * CHUNK, CHUNK)]
        pltpu.async_copy(x_hbm.at[idx_ref], bufs[k], gsems[k])

    def wait_g(k):
        pltpu.make_async_copy(x_hbm.at[perm_v.at[pl.ds(0, CHUNK)]], bufs[k],
                              gsems[k]).wait()

    def issue_w(c, k):
        @pl.when(c == nfc - 1)
        def _():
            def fixr(r, carry):
                pltpu.sync_copy(zeros_hbm.at[pl.ds(0, 1)],
                                bufs[k].at[pl.ds(r, 1)])
                return carry

            lax.fori_loop(f - c * CHUNK, CHUNK, fixr, 0)

        pltpu.async_copy(bufs[k], out_hbm.at[pl.ds(lo + c * CHUNK, CHUNK)],
                         wsems[k])

    def wait_w(k):
        pltpu.make_async_copy(zeros_hbm, bufs[k], wsems[k]).wait()

    for k in range(NBUF):
        @pl.when(k < nfc)
        def _(k=k):
            issue_g(k, k)

    def body_a(p, carry):
        for k in range(NBUF):
            c = NBUF * p + k

            @pl.when(c < nfc)
            def _(c=c, k=k):
                wait_g(k)
                issue_w(c, k)

            @pl.when(c + NBUF < nfc)
            def _(c=c, k=k):
                wait_w(k)
                issue_g(c + NBUF, k)

        return carry

    lax.fori_loop(0, (NCH + NBUF - 1) // NBUF, body_a, 0)

    for k in range(NBUF):
        @pl.when(k < nfc)
        def _(k=k):
            wait_w(k)

    @pl.when(nfc < NCH)
    def _():
        pltpu.sync_copy(zeros_hbm, stage_a)

        def body_b(c, carry):
            pltpu.async_copy(stage_a, out_hbm.at[pl.ds(lo + c * CHUNK, CHUNK)],
                             sem_wb)
            return carry

        lax.fori_loop(nfc, NCH, body_b, 0)

        def drain_b(c, carry):
            pltpu.make_async_copy(zeros_hbm, stage_a, sem_wb).wait()
            return carry

        lax.fori_loop(nfc, NCH, drain_b, 0)


@functools.cache
def _make_dispatch():
    return functools.partial(
        pl.kernel,
        out_type=jax.ShapeDtypeStruct((NROWS, D), jnp.float32),
        mesh=plsc.VectorSubcoreMesh(core_axis_name="c", subcore_axis_name="s"),
        compiler_params=pltpu.CompilerParams(needs_layout_passes=False),
        scratch_types=[
            pltpu.VMEM((NTOK,), jnp.int32),
            pltpu.VMEM((512,), jnp.int32),
            pltpu.VMEM((CHUNK, D), jnp.float32),
            pltpu.VMEM((CHUNK, D), jnp.float32),
            pltpu.VMEM((CHUNK, D), jnp.float32),
            pltpu.SemaphoreType.DMA,
            pltpu.SemaphoreType.DMA,
            pltpu.SemaphoreType.DMA,
            pltpu.SemaphoreType.DMA,
            pltpu.SemaphoreType.DMA,
            pltpu.SemaphoreType.DMA,
        ],
    )(_dispatch_body)


def kernel(x, W, b):
    xf = x.reshape(-1, D)
    slots = _router(xf, W, b.reshape(1, NE)).reshape(NTOK)
    zrows = jnp.zeros((CHUNK, D), jnp.float32)
    return _make_dispatch()(xf, slots, zrows)


# speedup vs baseline: 2.2486x; 1.0546x over previous
import functools

import jax
import jax.numpy as jnp
from jax import lax
from jax.experimental import pallas as pl
from jax.experimental.pallas import tpu as pltpu
from jax.experimental.pallas import tpu_sc as plsc

NE = 8
CAP = 1280
NTOK = 8192
D = 2048
BLK = 1024
NBLK = NTOK // BLK
NROWS = NE * CAP
NW = 32
RPT = NROWS // NW
CHUNK = 16
NCH = RPT // CHUNK
DROP = NROWS


def _router_body(x_ref, w_ref, b_ref, slot_ref, counts_ref, tril_ref):
    i = pl.program_id(0)

    @pl.when(i == 0)
    def _():
        counts_ref[...] = jnp.zeros((1, NE), jnp.float32)
        row_i = lax.broadcasted_iota(jnp.int32, (BLK, BLK), 0)
        col_i = lax.broadcasted_iota(jnp.int32, (BLK, BLK), 1)
        tril_ref[...] = (row_i >= col_i).astype(jnp.float32)

    logits = (
        jnp.dot(x_ref[...], w_ref[...], preferred_element_type=jnp.float32)
        + b_ref[...]
    )
    iota_e = lax.broadcasted_iota(jnp.int32, (BLK, NE), 1)
    mx = jnp.max(logits, axis=1, keepdims=True)
    routes = jnp.min(
        jnp.where(logits == mx, iota_e, NE), axis=1, keepdims=True
    )
    one_hot = (iota_e == routes).astype(jnp.float32)
    pos_incl = jnp.dot(tril_ref[...], one_hot,
                       preferred_element_type=jnp.float32)
    counts = counts_ref[...]
    pos_tok = jnp.sum(one_hot * (pos_incl + counts), axis=1, keepdims=True) - 1.0
    pos_i = pos_tok.astype(jnp.int32)
    keep = pos_i < CAP
    slot = jnp.where(keep, routes * CAP + pos_i, DROP)
    slot_ref[...] = slot.reshape(1, BLK, 1)
    counts_ref[...] = counts + jnp.sum(one_hot, axis=0, keepdims=True)


_router = pl.pallas_call(
    _router_body,
    grid=(NBLK,),
    in_specs=[
        pl.BlockSpec((BLK, D), lambda i: (i, 0)),
        pl.BlockSpec((D, NE), lambda i: (0, 0)),
        pl.BlockSpec((1, NE), lambda i: (0, 0)),
    ],
    out_specs=pl.BlockSpec((1, BLK, 1), lambda i: (i, 0, 0)),
    out_shape=jax.ShapeDtypeStruct((NBLK, BLK, 1), jnp.int32),
    scratch_shapes=[
        pltpu.VMEM((1, NE), jnp.float32),
        pltpu.VMEM((BLK, BLK), jnp.float32),
    ],
)


def _dispatch_body(x_hbm, slots_hbm, zeros_hbm, out_hbm,
                   slots_v, perm_v, stage_a, stage_b, zero_v, sem_g, sem_w):
    wid = lax.axis_index("s") * 2 + lax.axis_index("c")
    lo = wid * RPT
    pltpu.sync_copy(slots_hbm, slots_v)
    pltpu.sync_copy(zeros_hbm, zero_v)

    def initp(j, carry):
        perm_v[pl.ds(j * 16, 16)] = jnp.zeros((16,), jnp.int32)
        return carry

    lax.fori_loop(0, RPT // 16, initp, 0)

    lov = jnp.full((16,), lo, jnp.int32)
    lane = lax.iota(jnp.int32, 16)
    UNROLL = 8

    def build(jj, fcar):
        base = jj * (16 * UNROLL)
        acc = fcar
        for u in range(UNROLL):
            s = slots_v[pl.ds(base + u * 16, 16)]
            tok = lane + (base + u * 16)
            rel = s - lov
            m = (rel >= 0) & (rel < RPT)
            plsc.store_scatter(perm_v, [rel], tok, mask=m)
            acc = acc + plsc.all_reduce_population_count(m)
        return acc

    fvec = lax.fori_loop(0, NTOK // (16 * UNROLL), build,
                         jnp.zeros((16,), jnp.int32))
    f = jnp.max(fvec, axis=0)
    nfc = (f + CHUNK - 1) // CHUNK

    def issue_g(c, buf):
        idx_ref = perm_v.at[pl.ds(c * CHUNK, CHUNK)]
        pltpu.async_copy(x_hbm.at[idx_ref], buf, sem_g)

    def wait_g():
        pltpu.make_async_copy(x_hbm.at[perm_v.at[pl.ds(0, CHUNK)]], stage_a,
                              sem_g).wait()

    def issue_w(c, buf):
        @pl.when(c == nfc - 1)
        def _():
            def fixr(r, carry):
                pltpu.sync_copy(zeros_hbm.at[pl.ds(0, 1)],
                                buf.at[pl.ds(r, 1)])
                return carry

            lax.fori_loop(f - c * CHUNK, CHUNK, fixr, 0)

        pltpu.async_copy(buf, out_hbm.at[pl.ds(lo + c * CHUNK, CHUNK)], sem_w)

    def wait_w():
        pltpu.make_async_copy(zeros_hbm, stage_a, sem_w).wait()

    @pl.when(nfc > 0)
    def _():
        issue_g(0, stage_a)

    def body_a(p, carry):
        a = 2 * p
        b = 2 * p + 1

        @pl.when(a < nfc)
        def _():
            wait_g()

            @pl.when(a >= 1)
            def _():
                wait_w()

            @pl.when(b < nfc)
            def _():
                issue_g(b, stage_b)

            issue_w(a, stage_a)

        @pl.when(b < nfc)
        def _():
            wait_g()

            @pl.when(b + 1 < nfc)
            def _():
                issue_g(b + 1, stage_a)

            wait_w()
            issue_w(b, stage_b)

        return carry

    lax.fori_loop(0, (NCH + 1) // 2, body_a, 0)

    @pl.when(nfc > 0)
    def _():
        wait_w()

    def body_b(c, carry):
        pltpu.async_copy(zero_v, out_hbm.at[pl.ds(lo + c * CHUNK, CHUNK)],
                         sem_w)
        return carry

    lax.fori_loop(nfc, NCH, body_b, 0)

    def drain_b(c, carry):
        pltpu.make_async_copy(zeros_hbm, stage_a, sem_w).wait()
        return carry

    lax.fori_loop(nfc, NCH, drain_b, 0)


@functools.cache
def _make_dispatch():
    return functools.partial(
        pl.kernel,
        out_type=jax.ShapeDtypeStruct((NROWS, D), jnp.float32),
        mesh=plsc.VectorSubcoreMesh(core_axis_name="c", subcore_axis_name="s"),
        compiler_params=pltpu.CompilerParams(needs_layout_passes=False),
        scratch_types=[
            pltpu.VMEM((NTOK,), jnp.int32),
            pltpu.VMEM((512,), jnp.int32),
            pltpu.VMEM((CHUNK, D), jnp.float32),
            pltpu.VMEM((CHUNK, D), jnp.float32),
            pltpu.VMEM((CHUNK, D), jnp.float32),
            pltpu.SemaphoreType.DMA,
            pltpu.SemaphoreType.DMA,
        ],
    )(_dispatch_body)


def kernel(x, W, b):
    xf = x.reshape(-1, D)
    slots = _router(xf, W, b.reshape(1, NE)).reshape(NTOK)
    zrows = jnp.zeros((CHUNK, D), jnp.float32)
    return _make_dispatch()(xf, slots, zrows)
